# baseline (speedup 1.0000x reference)
import jax
import jax.numpy as jnp
from jax.experimental import pallas as pl
from jax.experimental.pallas import tpu as pltpu

NUM_HEADS = 16
NUM_KV_HEADS = 4
HEAD_DIM = 128
SCALE = 0.08838834764831845
BQ = 256
NEG = -1e30


def _flash_body(q_ref, k_ref, v_ref, o_ref):
    qi = pl.program_id(1)
    D = HEAD_DIM
    rep = NUM_HEADS // NUM_KV_HEADS
    r = jax.lax.broadcasted_iota(jnp.int32, (BQ, BQ), 0)
    c = jax.lax.broadcasted_iota(jnp.int32, (BQ, BQ), 1)
    tri = r >= c

    for h in range(NUM_HEADS):
        g = h // rep
        qh = q_ref[0, :, h * D:(h + 1) * D]

        def step(ki, carry, qh=qh, g=g):
            m, l, acc = carry
            kk = k_ref[0, pl.ds(ki * BQ, BQ), g * D:(g + 1) * D]
            s = jax.lax.dot_general(qh, kk, (((1,), (1,)), ((), ())),
                                    preferred_element_type=jnp.float32) * SCALE
            m_new = jnp.maximum(m, s.max(axis=1, keepdims=True))
            p = jnp.exp(s - m_new)
            alpha = jnp.exp(m - m_new)
            vv = v_ref[0, pl.ds(ki * BQ, BQ), g * D:(g + 1) * D]
            acc = acc * alpha + jax.lax.dot_general(
                p.astype(jnp.bfloat16), vv, (((1,), (0,)), ((), ())),
                preferred_element_type=jnp.float32)
            l = l * alpha + p.sum(axis=1, keepdims=True)
            return m_new, l, acc

        m0 = jnp.full((BQ, 1), NEG, jnp.float32)
        l0 = jnp.zeros((BQ, 1), jnp.float32)
        acc0 = jnp.zeros((BQ, D), jnp.float32)
        m, l, acc = jax.lax.fori_loop(0, qi, step, (m0, l0, acc0))

        kk = k_ref[0, pl.ds(qi * BQ, BQ), g * D:(g + 1) * D]
        s = jax.lax.dot_general(qh, kk, (((1,), (1,)), ((), ())),
                                preferred_element_type=jnp.float32) * SCALE
        s = jnp.where(tri, s, NEG)
        m_new = jnp.maximum(m, s.max(axis=1, keepdims=True))
        p = jnp.exp(s - m_new)
        alpha = jnp.exp(m - m_new)
        vv = v_ref[0, pl.ds(qi * BQ, BQ), g * D:(g + 1) * D]
        acc = acc * alpha + jax.lax.dot_general(
            p.astype(jnp.bfloat16), vv, (((1,), (0,)), ((), ())),
            preferred_element_type=jnp.float32)
        l = l * alpha + p.sum(axis=1, keepdims=True)

        o_ref[0, :, h * D:(h + 1) * D] = acc / l


def kernel(q, k, v, cu_seqlens_q):
    B = int(cu_seqlens_q.shape[0]) - 1
    T = q.shape[0]
    S = T // B
    nq = S // BQ
    HD = NUM_HEADS * HEAD_DIM
    GD = NUM_KV_HEADS * HEAD_DIM

    qr = q.reshape(B, S, HD).astype(jnp.bfloat16)
    kr = k.reshape(B, S, GD).astype(jnp.bfloat16)
    vr = v.reshape(B, S, GD).astype(jnp.bfloat16)

    ob = pl.pallas_call(
        _flash_body,
        grid=(B, nq),
        in_specs=[
            pl.BlockSpec((1, BQ, HD), lambda b, i: (b, i, 0)),
            pl.BlockSpec((1, S, GD), lambda b, i: (b, 0, 0)),
            pl.BlockSpec((1, S, GD), lambda b, i: (b, 0, 0)),
        ],
        out_specs=pl.BlockSpec((1, BQ, HD), lambda b, i: (b, i, 0)),
        out_shape=jax.ShapeDtypeStruct((B, S, HD), jnp.float32),
        compiler_params=pltpu.CompilerParams(
            dimension_semantics=("parallel", "arbitrary")),
    )(qr, kr, vr)

    return ob.reshape(T, NUM_HEADS, HEAD_DIM)


# speedup vs baseline: 1.4955x; 1.4955x over previous
import jax
import jax.numpy as jnp
from jax.experimental import pallas as pl
from jax.experimental.pallas import tpu as pltpu

NUM_HEADS = 16
NUM_KV_HEADS = 4
HEAD_DIM = 128
SCALE = 0.08838834764831845
BQ = 256
GQ = 4 * BQ
NEG = -1e30


def _qk(qs, k_ref, g, ki):
    D = HEAD_DIM
    kk = k_ref[0, pl.ds(ki * BQ, BQ), g * D:(g + 1) * D]
    return jax.lax.dot_general(qs, kk, (((1,), (1,)), ((), ())),
                               preferred_element_type=jnp.float32)


def _flash_body(q_ref, k_ref, v_ref, o_ref):
    qi = pl.program_id(1)
    D = HEAD_DIM
    rep = NUM_HEADS // NUM_KV_HEADS
    r = jax.lax.broadcasted_iota(jnp.int32, (GQ, BQ), 0)
    c = jax.lax.broadcasted_iota(jnp.int32, (GQ, BQ), 1)
    tri = (r % BQ) >= c

    for g in range(NUM_KV_HEADS):
        qs = jnp.concatenate(
            [q_ref[0, :, (g * rep + j) * D:(g * rep + j + 1) * D]
             for j in range(rep)], axis=0)

        def update(carry, s, vv):
            m, l, acc = carry
            m_new = jnp.maximum(m, s.max(axis=1, keepdims=True))
            p = jnp.exp(s - m_new)
            alpha = jnp.exp(m - m_new)
            acc = acc * alpha + jax.lax.dot_general(
                p.astype(jnp.bfloat16), vv, (((1,), (0,)), ((), ())),
                preferred_element_type=jnp.float32)
            l = l * alpha + p.sum(axis=1, keepdims=True)
            return m_new, l, acc

        def step(ki, carry, qs=qs, g=g):
            m, l, acc, s = carry
            s_next = _qk(qs, k_ref, g, ki + 1)
            vv = v_ref[0, pl.ds(ki * BQ, BQ), g * D:(g + 1) * D]
            m, l, acc = update((m, l, acc), s, vv)
            return m, l, acc, s_next

        m0 = jnp.full((GQ, 1), NEG, jnp.float32)
        l0 = jnp.zeros((GQ, 1), jnp.float32)
        acc0 = jnp.zeros((GQ, D), jnp.float32)
        s0 = _qk(qs, k_ref, g, 0)
        m, l, acc, s = jax.lax.fori_loop(0, qi, step, (m0, l0, acc0, s0))

        s = jnp.where(tri, s, NEG)
        vv = v_ref[0, pl.ds(qi * BQ, BQ), g * D:(g + 1) * D]
        m, l, acc = update((m, l, acc), s, vv)

        out = acc / l
        for j in range(rep):
            h = g * rep + j
            o_ref[0, :, h * D:(h + 1) * D] = out[j * BQ:(j + 1) * BQ]


def kernel(q, k, v, cu_seqlens_q):
    B = int(cu_seqlens_q.shape[0]) - 1
    T = q.shape[0]
    S = T // B
    nq = S // BQ
    HD = NUM_HEADS * HEAD_DIM
    GD = NUM_KV_HEADS * HEAD_DIM

    qr = (q.reshape(B, S, HD) * SCALE).astype(jnp.bfloat16)
    kr = k.reshape(B, S, GD).astype(jnp.bfloat16)
    vr = v.reshape(B, S, GD).astype(jnp.bfloat16)

    ob = pl.pallas_call(
        _flash_body,
        grid=(B, nq),
        in_specs=[
            pl.BlockSpec((1, BQ, HD), lambda b, i: (b, i, 0)),
            pl.BlockSpec((1, S, GD), lambda b, i: (b, 0, 0)),
            pl.BlockSpec((1, S, GD), lambda b, i: (b, 0, 0)),
        ],
        out_specs=pl.BlockSpec((1, BQ, HD), lambda b, i: (b, i, 0)),
        out_shape=jax.ShapeDtypeStruct((B, S, HD), jnp.float32),
        compiler_params=pltpu.CompilerParams(
            dimension_semantics=("parallel", "arbitrary")),
    )(qr, kr, vr)

    return ob.reshape(T, NUM_HEADS, HEAD_DIM)


# speedup vs baseline: 1.5182x; 1.0152x over previous
import jax
import jax.numpy as jnp
from jax.experimental import pallas as pl
from jax.experimental.pallas import tpu as pltpu

NUM_HEADS = 16
NUM_KV_HEADS = 4
HEAD_DIM = 128
SCALE = 0.08838834764831845
BQ = 256
GQ = 4 * BQ
NEG = -1e30


def _qk(qs, k_ref, g, ki):
    D = HEAD_DIM
    kk = k_ref[0, pl.ds(ki * BQ, BQ), g * D:(g + 1) * D]
    return jax.lax.dot_general(qs, kk, (((1,), (1,)), ((), ())),
                               preferred_element_type=jnp.float32)


def _flash_body(q_ref, k_ref, v_ref, o_ref):
    qi = pl.program_id(1)
    D = HEAD_DIM
    rep = NUM_HEADS // NUM_KV_HEADS
    r = jax.lax.broadcasted_iota(jnp.int32, (GQ, BQ), 0)
    c = jax.lax.broadcasted_iota(jnp.int32, (GQ, BQ), 1)
    tri = (r % BQ) >= c

    for g in range(NUM_KV_HEADS):
        qs = jnp.concatenate(
            [q_ref[0, :, (g * rep + j) * D:(g * rep + j + 1) * D]
             for j in range(rep)], axis=0)

        def update(carry, s, vv):
            m, l, acc = carry
            m_new = jnp.maximum(m, s.max(axis=1, keepdims=True))
            p = jnp.exp2(s - m_new)
            alpha = jnp.exp2(m - m_new)
            acc = acc * alpha + jax.lax.dot_general(
                p.astype(jnp.bfloat16), vv, (((1,), (0,)), ((), ())),
                preferred_element_type=jnp.float32)
            l = l * alpha + p.sum(axis=1, keepdims=True)
            return m_new, l, acc

        def step(ki, carry, qs=qs, g=g):
            m, l, acc, s = carry
            s_next = _qk(qs, k_ref, g, ki + 1)
            vv = v_ref[0, pl.ds(ki * BQ, BQ), g * D:(g + 1) * D]
            m, l, acc = update((m, l, acc), s, vv)
            return m, l, acc, s_next

        m0 = jnp.full((GQ, 1), NEG, jnp.float32)
        l0 = jnp.zeros((GQ, 1), jnp.float32)
        acc0 = jnp.zeros((GQ, D), jnp.float32)
        s0 = _qk(qs, k_ref, g, 0)
        m, l, acc, s = jax.lax.fori_loop(0, qi, step, (m0, l0, acc0, s0))

        s = jnp.where(tri, s, NEG)
        vv = v_ref[0, pl.ds(qi * BQ, BQ), g * D:(g + 1) * D]
        m, l, acc = update((m, l, acc), s, vv)

        out = acc / l
        for j in range(rep):
            h = g * rep + j
            o_ref[0, :, h * D:(h + 1) * D] = out[j * BQ:(j + 1) * BQ]


def kernel(q, k, v, cu_seqlens_q):
    B = int(cu_seqlens_q.shape[0]) - 1
    T = q.shape[0]
    S = T // B
    nq = S // BQ
    HD = NUM_HEADS * HEAD_DIM
    GD = NUM_KV_HEADS * HEAD_DIM

    qr = (q.reshape(B, S, HD) * (SCALE * 1.4426950408889634)).astype(jnp.bfloat16)
    kr = k.reshape(B, S, GD).astype(jnp.bfloat16)
    vr = v.reshape(B, S, GD).astype(jnp.bfloat16)

    ob = pl.pallas_call(
        _flash_body,
        grid=(B, nq),
        in_specs=[
            pl.BlockSpec((1, BQ, HD), lambda b, i: (b, i, 0)),
            pl.BlockSpec((1, S, GD), lambda b, i: (b, 0, 0)),
            pl.BlockSpec((1, S, GD), lambda b, i: (b, 0, 0)),
        ],
        out_specs=pl.BlockSpec((1, BQ, HD), lambda b, i: (b, i, 0)),
        out_shape=jax.ShapeDtypeStruct((B, S, HD), jnp.float32),
        compiler_params=pltpu.CompilerParams(
            dimension_semantics=("parallel", "arbitrary")),
    )(qr, kr, vr)

    return ob.reshape(T, NUM_HEADS, HEAD_DIM)
